# baseline (device time: 27504 ns/iter reference)
import jax
import jax.numpy as jnp
from jax import lax
from jax.experimental import pallas as pl
from jax.experimental.pallas import tpu as pltpu

N_DEV = 32
N_TOK = 512
D_IN = 256
D_OUT = 512
N_EXP = 128
N_EXP_LOCAL = N_EXP // N_DEV
CH = N_TOK // N_DEV


def kernel(x, router_W, route_idx, expert_W, shared_W):
    def body(x_ref, rW_ref, idx_ref, eW_ref, sW_ref, out_ref,
             acc_ref, rs_buf, ag_buf, rs_send, rs_recv, ag_send, ag_recv):
        pos = lax.axis_index("i")

        barrier = pltpu.get_barrier_semaphore()
        for c in range(N_DEV):
            @pl.when(c != pos)
            def _():
                pl.semaphore_signal(
                    barrier, inc=1,
                    device_id=(c,), device_id_type=pl.DeviceIdType.MESH,
                )
        pl.semaphore_wait(barrier, N_DEV - 1)

        xb = x_ref[...].astype(jnp.bfloat16)

        scores = jnp.dot(xb, rW_ref[...].astype(jnp.bfloat16),
                         preferred_element_type=jnp.float32)
        mx = jnp.max(scores, axis=-1, keepdims=True)
        ex = jnp.exp(scores - mx)
        probs = ex / jnp.sum(ex, axis=-1, keepdims=True)
        ridx = idx_ref[...]
        cols = lax.broadcasted_iota(jnp.int32, (N_TOK, N_EXP), 1)
        p_top = jnp.sum(jnp.where(cols == ridx, probs, 0.0),
                        axis=1, keepdims=True)

        partial = jnp.zeros((N_TOK, D_OUT), jnp.float32)
        for j in range(N_EXP_LOCAL):
            eid = pos * N_EXP_LOCAL + j
            scale = jnp.where(ridx == eid, p_top, 0.0)
            xw = jnp.dot(xb, eW_ref[j].astype(jnp.bfloat16),
                         preferred_element_type=jnp.float32)
            partial = partial + scale * xw
        acc_ref[...] = partial.astype(jnp.bfloat16)

        for c in range(N_DEV):
            @pl.when(c != pos)
            def _():
                pltpu.make_async_remote_copy(
                    src_ref=acc_ref.at[pl.ds(c * CH, CH), :],
                    dst_ref=rs_buf.at[pos],
                    send_sem=rs_send.at[c],
                    recv_sem=rs_recv.at[pos],
                    device_id=(c,),
                    device_id_type=pl.DeviceIdType.MESH,
                ).start()

        shared = jnp.dot(xb, sW_ref[...].astype(jnp.bfloat16),
                         preferred_element_type=jnp.float32)

        rs_buf[pl.ds(pos, 1)] = acc_ref[pl.ds(pos * CH, CH), :][None]

        for s in range(N_DEV):
            @pl.when(s != pos)
            def _():
                pltpu.make_async_remote_copy(
                    src_ref=rs_buf.at[s], dst_ref=rs_buf.at[s],
                    send_sem=rs_send.at[s], recv_sem=rs_recv.at[s],
                    device_id=(s,), device_id_type=pl.DeviceIdType.MESH,
                ).wait_recv()
        reduced = jnp.sum(rs_buf[...].astype(jnp.float32), axis=0)

        ag_buf[pl.ds(pos, 1)] = reduced.astype(jnp.bfloat16)[None]
        for c in range(N_DEV):
            @pl.when(c != pos)
            def _():
                pltpu.make_async_remote_copy(
                    src_ref=ag_buf.at[pos], dst_ref=ag_buf.at[pos],
                    send_sem=ag_send.at[c], recv_sem=ag_recv.at[pos],
                    device_id=(c,), device_id_type=pl.DeviceIdType.MESH,
                ).start()

        for c in range(N_DEV):
            @pl.when(c != pos)
            def _():
                pltpu.make_async_remote_copy(
                    src_ref=acc_ref.at[pl.ds(c * CH, CH), :],
                    dst_ref=rs_buf.at[pos],
                    send_sem=rs_send.at[c], recv_sem=rs_recv.at[pos],
                    device_id=(c,), device_id_type=pl.DeviceIdType.MESH,
                ).wait_send()

        for s in range(N_DEV):
            @pl.when(s != pos)
            def _():
                pltpu.make_async_remote_copy(
                    src_ref=ag_buf.at[s], dst_ref=ag_buf.at[s],
                    send_sem=ag_send.at[s], recv_sem=ag_recv.at[s],
                    device_id=(s,), device_id_type=pl.DeviceIdType.MESH,
                ).wait_recv()

        for c in range(N_DEV):
            @pl.when(c != pos)
            def _():
                pltpu.make_async_remote_copy(
                    src_ref=ag_buf.at[pos], dst_ref=ag_buf.at[pos],
                    send_sem=ag_send.at[c], recv_sem=ag_recv.at[pos],
                    device_id=(c,), device_id_type=pl.DeviceIdType.MESH,
                ).wait_send()

        out_ref[...] = shared + ag_buf[...].astype(jnp.float32).reshape(
            N_TOK, D_OUT)

    return pl.pallas_call(
        body,
        out_shape=jax.ShapeDtypeStruct((N_TOK, D_OUT), jnp.float32),
        in_specs=[pl.BlockSpec(memory_space=pltpu.VMEM)] * 5,
        out_specs=pl.BlockSpec(memory_space=pltpu.VMEM),
        scratch_shapes=[
            pltpu.VMEM((N_TOK, D_OUT), jnp.bfloat16),
            pltpu.VMEM((N_DEV, CH, D_OUT), jnp.bfloat16),
            pltpu.VMEM((N_DEV, CH, D_OUT), jnp.bfloat16),
            pltpu.SemaphoreType.DMA((N_DEV,)),
            pltpu.SemaphoreType.DMA((N_DEV,)),
            pltpu.SemaphoreType.DMA((N_DEV,)),
            pltpu.SemaphoreType.DMA((N_DEV,)),
        ],
        compiler_params=pltpu.CompilerParams(collective_id=0),
    )(x, router_W, route_idx, expert_W, shared_W)


# device time: 26864 ns/iter; 1.0238x vs baseline; 1.0238x over previous
import jax
import jax.numpy as jnp
from jax import lax
from jax.experimental import pallas as pl
from jax.experimental.pallas import tpu as pltpu

N_DEV = 32
N_TOK = 512
D_IN = 256
D_OUT = 512
N_EXP = 128
N_EXP_LOCAL = N_EXP // N_DEV
CH = N_TOK // N_DEV


def kernel(x, router_W, route_idx, expert_W, shared_W):
    def body(x_ref, rW_ref, idx_ref, eW_ref, sW_ref, out_ref,
             acc_ref, rs_buf, ag_buf, rs_send, rs_recv, ag_send, ag_recv):
        pos = lax.axis_index("i")

        rs_buf[...] = jnp.zeros((N_DEV, CH, D_OUT), jnp.bfloat16)

        barrier = pltpu.get_barrier_semaphore()
        for c in range(N_DEV):
            @pl.when(c != pos)
            def _():
                pl.semaphore_signal(
                    barrier, inc=1,
                    device_id=(c,), device_id_type=pl.DeviceIdType.MESH,
                )

        xb = x_ref[...].astype(jnp.bfloat16)

        scores = jnp.dot(xb, rW_ref[...].astype(jnp.bfloat16),
                         preferred_element_type=jnp.float32)
        mx = jnp.max(scores, axis=-1, keepdims=True)
        ex = jnp.exp(scores - mx)
        probs = ex / jnp.sum(ex, axis=-1, keepdims=True)
        ridx = idx_ref[...]
        cols = lax.broadcasted_iota(jnp.int32, (N_TOK, N_EXP), 1)
        p_top = jnp.sum(jnp.where(cols == ridx, probs, 0.0),
                        axis=1, keepdims=True)

        partial = jnp.zeros((N_TOK, D_OUT), jnp.float32)
        for j in range(N_EXP_LOCAL):
            eid = pos * N_EXP_LOCAL + j
            scale = jnp.where(ridx == eid, p_top, 0.0)
            xw = jnp.dot(xb, eW_ref[j].astype(jnp.bfloat16),
                         preferred_element_type=jnp.float32)
            partial = partial + scale * xw
        acc_ref[...] = partial.astype(jnp.bfloat16)

        owner = ridx // N_EXP_LOCAL
        send_cnt = jnp.sum((owner == pos).astype(jnp.int32).reshape(N_DEV, CH),
                           axis=1, keepdims=True)
        rows = lax.broadcasted_iota(jnp.int32, (N_TOK, 1), 0)
        in_my_chunk = (rows >= pos * CH) & (rows < (pos + 1) * CH)
        peer_ids = lax.broadcasted_iota(jnp.int32, (N_TOK, N_DEV), 1)
        recv_cnt = jnp.sum(((owner == peer_ids) & in_my_chunk)
                           .astype(jnp.int32), axis=0, keepdims=True)

        pl.semaphore_wait(barrier, N_DEV - 1)
        for c in range(N_DEV):
            @pl.when((c != pos) & (send_cnt[c, 0] > 0))
            def _():
                pltpu.make_async_remote_copy(
                    src_ref=acc_ref.at[pl.ds(c * CH, CH), :],
                    dst_ref=rs_buf.at[pos],
                    send_sem=rs_send.at[c],
                    recv_sem=rs_recv.at[pos],
                    device_id=(c,),
                    device_id_type=pl.DeviceIdType.MESH,
                ).start()

        shared = jnp.dot(xb, sW_ref[...].astype(jnp.bfloat16),
                         preferred_element_type=jnp.float32)

        rs_buf[pl.ds(pos, 1)] = acc_ref[pl.ds(pos * CH, CH), :][None]

        for s in range(N_DEV):
            @pl.when((s != pos) & (recv_cnt[0, s] > 0))
            def _():
                pltpu.make_async_remote_copy(
                    src_ref=rs_buf.at[s], dst_ref=rs_buf.at[s],
                    send_sem=rs_send.at[s], recv_sem=rs_recv.at[s],
                    device_id=(s,), device_id_type=pl.DeviceIdType.MESH,
                ).wait_recv()
        reduced = jnp.sum(rs_buf[...].astype(jnp.float32), axis=0)

        ag_buf[pl.ds(pos, 1)] = reduced.astype(jnp.bfloat16)[None]
        for c in range(N_DEV):
            @pl.when(c != pos)
            def _():
                pltpu.make_async_remote_copy(
                    src_ref=ag_buf.at[pos], dst_ref=ag_buf.at[pos],
                    send_sem=ag_send.at[c], recv_sem=ag_recv.at[pos],
                    device_id=(c,), device_id_type=pl.DeviceIdType.MESH,
                ).start()

        for c in range(N_DEV):
            @pl.when((c != pos) & (send_cnt[c, 0] > 0))
            def _():
                pltpu.make_async_remote_copy(
                    src_ref=acc_ref.at[pl.ds(c * CH, CH), :],
                    dst_ref=rs_buf.at[pos],
                    send_sem=rs_send.at[c], recv_sem=rs_recv.at[pos],
                    device_id=(c,), device_id_type=pl.DeviceIdType.MESH,
                ).wait_send()

        for s in range(N_DEV):
            @pl.when(s != pos)
            def _():
                pltpu.make_async_remote_copy(
                    src_ref=ag_buf.at[s], dst_ref=ag_buf.at[s],
                    send_sem=ag_send.at[s], recv_sem=ag_recv.at[s],
                    device_id=(s,), device_id_type=pl.DeviceIdType.MESH,
                ).wait_recv()

        for c in range(N_DEV):
            @pl.when(c != pos)
            def _():
                pltpu.make_async_remote_copy(
                    src_ref=ag_buf.at[pos], dst_ref=ag_buf.at[pos],
                    send_sem=ag_send.at[c], recv_sem=ag_recv.at[pos],
                    device_id=(c,), device_id_type=pl.DeviceIdType.MESH,
                ).wait_send()

        out_ref[...] = shared + ag_buf[...].astype(jnp.float32).reshape(
            N_TOK, D_OUT)

    return pl.pallas_call(
        body,
        out_shape=jax.ShapeDtypeStruct((N_TOK, D_OUT), jnp.float32),
        in_specs=[pl.BlockSpec(memory_space=pltpu.VMEM)] * 5,
        out_specs=pl.BlockSpec(memory_space=pltpu.VMEM),
        scratch_shapes=[
            pltpu.VMEM((N_TOK, D_OUT), jnp.bfloat16),
            pltpu.VMEM((N_DEV, CH, D_OUT), jnp.bfloat16),
            pltpu.VMEM((N_DEV, CH, D_OUT), jnp.bfloat16),
            pltpu.SemaphoreType.DMA((N_DEV,)),
            pltpu.SemaphoreType.DMA((N_DEV,)),
            pltpu.SemaphoreType.DMA((N_DEV,)),
            pltpu.SemaphoreType.DMA((N_DEV,)),
        ],
        compiler_params=pltpu.CompilerParams(collective_id=0),
    )(x, router_W, route_idx, expert_W, shared_W)


# device time: 26270 ns/iter; 1.0470x vs baseline; 1.0226x over previous
import os

import jax
import jax.numpy as jnp
from jax import lax
from jax.experimental import pallas as pl
from jax.experimental.pallas import tpu as pltpu

_ABLATE = os.environ.get("MOE_ABLATE", "")

N_DEV = 32
N_TOK = 512
D_IN = 256
D_OUT = 512
N_EXP = 128
N_EXP_LOCAL = N_EXP // N_DEV
CH = N_TOK // N_DEV
N_BLK = 8
BLK = N_TOK // N_BLK
CH_PER_BLK = BLK // CH
_BLK_OFFS = (4, 3, 5, 2, 6, 1, 7, 0)


def kernel(x, router_W, route_idx, expert_W, shared_W):
    do_barrier = _ABLATE != "compute"
    do_phase1 = _ABLATE in ("", "phase1")
    do_phase2 = _ABLATE == ""

    def body(x_ref, rW_ref, idx_ref, eW_ref, sW_ref,
             out_ref, acc_ref, rs_buf, ag_buf, pt_ref,
             rs_send, rs_recv, ag_send, ag_recv):
        pos = lax.axis_index("i")
        my_blk = pos // CH_PER_BLK

        ridx_all = idx_ref[...]
        owner = ridx_all // N_EXP_LOCAL
        send_cnt = jnp.sum((owner == pos).astype(jnp.int32)
                           .reshape(N_DEV, CH), axis=1, keepdims=True)
        rows = lax.broadcasted_iota(jnp.int32, (N_TOK, 1), 0)
        in_my_chunk = (rows >= pos * CH) & (rows < (pos + 1) * CH)
        peer_ids = lax.broadcasted_iota(jnp.int32, (N_TOK, N_DEV), 1)
        recv_cnt = jnp.sum(((owner == peer_ids) & in_my_chunk)
                           .astype(jnp.int32), axis=0, keepdims=True)

        chunk_ids = lax.broadcasted_iota(jnp.int32, (N_DEV, 1), 0)
        sbits = jnp.where(send_cnt > 0, 1 << (chunk_ids % 16), 0)
        slo = jnp.sum(jnp.where(chunk_ids < 16, sbits, 0))
        shi = jnp.sum(jnp.where(chunk_ids >= 16, sbits, 0))
        peer_row = lax.broadcasted_iota(jnp.int32, (1, N_DEV), 1)
        rbits = jnp.where(recv_cnt > 0, 1 << (peer_row % 16), 0)
        rlo = jnp.sum(jnp.where(peer_row < 16, rbits, 0))
        rhi = jnp.sum(jnp.where(peer_row >= 16, rbits, 0))

        def send_flag(c):
            return jnp.where(c < 16, (slo >> c) & 1, (shi >> (c - 16)) & 1)

        def recv_flag(s):
            return jnp.where(s < 16, (rlo >> s) & 1, (rhi >> (s - 16)) & 1)

        if do_phase1:
            rs_buf[...] = jnp.zeros((N_DEV, CH, D_OUT), jnp.bfloat16)

        if do_barrier:
            barrier = pltpu.get_barrier_semaphore()
            for c in range(N_DEV):
                @pl.when(c != pos)
                def _():
                    pl.semaphore_signal(
                        barrier, inc=1,
                        device_id=(c,), device_id_type=pl.DeviceIdType.MESH,
                    )

        xb = x_ref[...].astype(jnp.bfloat16)

        scores = jnp.dot(xb, rW_ref[...].astype(jnp.bfloat16),
                         preferred_element_type=jnp.float32)
        mx = jnp.max(scores, axis=-1, keepdims=True)
        ex = jnp.exp(scores - mx)
        probs = ex / jnp.sum(ex, axis=-1, keepdims=True)
        cols = lax.broadcasted_iota(jnp.int32, (N_TOK, N_EXP), 1)
        pt_ref[...] = jnp.sum(jnp.where(cols == ridx_all, probs, 0.0),
                              axis=1, keepdims=True)

        w_big = eW_ref[...].astype(jnp.bfloat16).reshape(
            N_EXP_LOCAL * D_IN, D_OUT)

        def send_block_chunks(blk):
            for i in range(CH_PER_BLK):
                c = blk * CH_PER_BLK + i
                @pl.when((c != pos) & (send_flag(c) > 0))
                def _():
                    pltpu.make_async_remote_copy(
                        src_ref=acc_ref.at[pl.ds(c * CH, CH), :],
                        dst_ref=rs_buf.at[pos],
                        send_sem=rs_send.at[c],
                        recv_sem=rs_recv.at[pos],
                        device_id=(c,),
                        device_id_type=pl.DeviceIdType.MESH,
                    ).start()

        blks = [(my_blk + off) % N_BLK for off in _BLK_OFFS]
        for k, blk in enumerate(blks):
            r0 = blk * BLK
            xblk = x_ref[pl.ds(r0, BLK), :].astype(jnp.bfloat16)
            ridx = idx_ref[pl.ds(r0, BLK), :]
            pt = pt_ref[pl.ds(r0, BLK), :]
            x_big = jnp.concatenate(
                [jnp.where(ridx == pos * N_EXP_LOCAL + j, pt, 0.0)
                 .astype(jnp.bfloat16) * xblk
                 for j in range(N_EXP_LOCAL)], axis=1)
            acc_ref[pl.ds(r0, BLK), :] = jnp.dot(
                x_big, w_big, preferred_element_type=jnp.float32,
            ).astype(jnp.bfloat16)

            if do_phase1:
                if k == 1:
                    if do_barrier:
                        pl.semaphore_wait(barrier, N_DEV - 1)
                    send_block_chunks(blks[0])
                if k >= 1:
                    send_block_chunks(blk)
        if do_barrier and not do_phase1:
            pl.semaphore_wait(barrier, N_DEV - 1)

        if do_phase1:
            rs_buf[pl.ds(pos, 1)] = acc_ref[pl.ds(pos * CH, CH), :][None]

            for s in range(N_DEV):
                @pl.when((s != pos) & (recv_flag(s) > 0))
                def _():
                    pltpu.make_async_remote_copy(
                        src_ref=rs_buf.at[s], dst_ref=rs_buf.at[s],
                        send_sem=rs_send.at[s], recv_sem=rs_recv.at[s],
                        device_id=(s,), device_id_type=pl.DeviceIdType.MESH,
                    ).wait_recv()
            reduced = jnp.sum(rs_buf[...].astype(jnp.float32), axis=0)

        if do_phase2:
            ag_buf[pl.ds(pos, 1)] = reduced.astype(jnp.bfloat16)[None]
            for c in range(N_DEV):
                @pl.when(c != pos)
                def _():
                    pltpu.make_async_remote_copy(
                        src_ref=ag_buf.at[pos], dst_ref=ag_buf.at[pos],
                        send_sem=ag_send.at[c], recv_sem=ag_recv.at[pos],
                        device_id=(c,), device_id_type=pl.DeviceIdType.MESH,
                    ).start()

        shared = jnp.dot(xb, sW_ref[...].astype(jnp.bfloat16),
                         preferred_element_type=jnp.float32)

        if do_phase1:
            for blk in blks:
                for i in range(CH_PER_BLK):
                    c = blk * CH_PER_BLK + i
                    @pl.when((c != pos) & (send_flag(c) > 0))
                    def _():
                        pltpu.make_async_remote_copy(
                            src_ref=acc_ref.at[pl.ds(c * CH, CH), :],
                            dst_ref=rs_buf.at[pos],
                            send_sem=rs_send.at[c], recv_sem=rs_recv.at[pos],
                            device_id=(c,),
                            device_id_type=pl.DeviceIdType.MESH,
                        ).wait_send()

        if do_phase2:
            out_ref[...] = shared

            for s in range(N_DEV):
                @pl.when(s != pos)
                def _():
                    pltpu.make_async_remote_copy(
                        src_ref=ag_buf.at[s], dst_ref=ag_buf.at[s],
                        send_sem=ag_send.at[s], recv_sem=ag_recv.at[s],
                        device_id=(s,), device_id_type=pl.DeviceIdType.MESH,
                    ).wait_recv()

            out_ref[...] = out_ref[...] + ag_buf[...].astype(
                jnp.float32).reshape(N_TOK, D_OUT)

            for c in range(N_DEV):
                @pl.when(c != pos)
                def _():
                    pltpu.make_async_remote_copy(
                        src_ref=ag_buf.at[pos], dst_ref=ag_buf.at[pos],
                        send_sem=ag_send.at[c], recv_sem=ag_recv.at[pos],
                        device_id=(c,), device_id_type=pl.DeviceIdType.MESH,
                    ).wait_send()
        elif do_phase1:
            out_ref[...] = shared
            out_ref[0:CH, :] = out_ref[0:CH, :] + reduced
        else:
            out_ref[...] = shared + acc_ref[...].astype(jnp.float32)

    return pl.pallas_call(
        body,
        out_shape=jax.ShapeDtypeStruct((N_TOK, D_OUT), jnp.float32),
        in_specs=[pl.BlockSpec(memory_space=pltpu.VMEM)] * 5,
        out_specs=pl.BlockSpec(memory_space=pltpu.VMEM),
        scratch_shapes=[
            pltpu.VMEM((N_TOK, D_OUT), jnp.bfloat16),
            pltpu.VMEM((N_DEV, CH, D_OUT), jnp.bfloat16),
            pltpu.VMEM((N_DEV, CH, D_OUT), jnp.bfloat16),
            pltpu.VMEM((N_TOK, 1), jnp.float32),
            pltpu.SemaphoreType.DMA((N_DEV,)),
            pltpu.SemaphoreType.DMA((N_DEV,)),
            pltpu.SemaphoreType.DMA((N_DEV,)),
            pltpu.SemaphoreType.DMA((N_DEV,)),
        ],
        compiler_params=(pltpu.CompilerParams(collective_id=0)
                         if do_barrier else pltpu.CompilerParams()),
    )(x, router_W, route_idx, expert_W, shared_W)
